# baseline (speedup 1.0000x reference)
import dataclasses
import functools

import jax
import jax.numpy as jnp
from jax import lax
from jax.experimental import pallas as pl
from jax.experimental.pallas import tpu as pltpu
from jax.experimental.pallas import tpu_sc as plsc

_N = 10000
_E = 320000
_D = 128
_NC = 2
_NS = 16
_LANES = 16
_NW = _NC * _NS
_EPW = _E // _NW
_K = 80
_CHUNKS = _EPW // _K
_K2 = 40
_CH2 = _EPW // _K2
_NP = 10240
_RPS = _NP // _NS
_ZR = 128

def _sc_mesh():
    return plsc.VectorSubcoreMesh(core_axis_name="c", subcore_axis_name="s")


def _agg_sc(m, src2, dst2):
    @functools.partial(
        pl.kernel,
        out_type=jax.ShapeDtypeStruct((_NC, _NP, _D), jnp.float32),
        mesh=_sc_mesh(),
        scratch_types=[
            pltpu.VMEM((1, _K2), jnp.int32),
            pltpu.VMEM((1, _K2), jnp.int32),
            pltpu.VMEM((1, _K2), jnp.int32),
            pltpu.VMEM((1, _K2), jnp.int32),
            pltpu.VMEM((1, _K2), jnp.int32),
            pltpu.VMEM((1, _K2), jnp.int32),
            pltpu.VMEM((1, _K2), jnp.int32),
            pltpu.VMEM((1, _K2), jnp.int32),
            pltpu.VMEM((_K2, _D), jnp.float32),
            pltpu.VMEM((_K2, _D), jnp.float32),
            pltpu.VMEM((_K2, _D), jnp.float32),
            pltpu.VMEM((_K2, _D), jnp.float32),
            pltpu.VMEM_SHARED((_NP, _D), jnp.float32),
            pltpu.SemaphoreType.DMA,
            pltpu.SemaphoreType.DMA,
            pltpu.SemaphoreType.DMA,
            pltpu.SemaphoreType.DMA,
            pltpu.SemaphoreType.DMA,
            pltpu.SemaphoreType.DMA,
            pltpu.SemaphoreType.DMA,
            pltpu.SemaphoreType.DMA,
            pltpu.SemaphoreType.DMA,
            pltpu.SemaphoreType.DMA,
            pltpu.SemaphoreType.DMA,
            pltpu.SemaphoreType.DMA,
            pltpu.SemaphoreType.DMA,
            pltpu.SemaphoreType.DMA,
            pltpu.SemaphoreType.DMA,
            pltpu.SemaphoreType.DMA,
        ],
    )
    def k(m_hbm, src_hbm, dst_hbm, out_hbm,
          sA, sB, sC, sD, dA, dB, dC, dD, r0, r1, r2, r3, agg_sh,
          g0, g1, g2, g3, s0, s1, s2, s3, e0, e1, e2, e3, f0, f1, f2, f3):
        cid = lax.axis_index("c")
        sid = lax.axis_index("s")
        w = cid * _NS + sid
        rows = (r0, r1, r2, r3)
        sbuf = (sA, sB, sC, sD)
        dbuf = (dA, dB, dC, dD)
        gsem = (g0, g1, g2, g3)
        ssem = (s0, s1, s2, s3)
        dsem = (e0, e1, e2, e3)
        fsem = (f0, f1, f2, f3)

        @pl.loop(0, _K2)
        def _(i):
            @pl.loop(0, _D // _LANES)
            def _(j):
                r0[i, pl.ds(j * _LANES, _LANES)] = jnp.zeros((_LANES,), jnp.float32)

        @pl.loop(0, _RPS // _K2)
        def _(t):
            pltpu.sync_copy(r0, agg_sh.at[pl.ds(sid * _RPS + t * _K2, _K2)])

        plsc.subcore_barrier()

        for t in (_CH2 - 2, _CH2 - 1):
            pltpu.sync_copy(src_hbm.at[w, t], sA)
            pltpu.async_copy(m_hbm.at[sA.at[0]], r0, g0).wait()
            pltpu.sync_copy(dst_hbm.at[w, t], dA)
            pltpu.sync_copy(r0, agg_sh.at[dA.at[0]], add=True)

        for b in range(4):
            pltpu.async_copy(src_hbm.at[w, b], sbuf[b], fsem[b])
            pltpu.async_copy(dst_hbm.at[w, b], dbuf[b], dsem[b])
        for b in range(4):
            pltpu.make_async_copy(src_hbm.at[w, b], sbuf[b], fsem[b]).wait()
            pltpu.async_copy(m_hbm.at[sbuf[b].at[0]], rows[b], gsem[b])

        @pl.loop(0, 61)
        def _(i):
            c = 4 * i
            for b in range(4):
                pltpu.make_async_copy(m_hbm.at[sbuf[b].at[0]], rows[b], gsem[b]).wait()
                pltpu.make_async_copy(dst_hbm.at[w, c + b], dbuf[b], dsem[b]).wait()
                pltpu.async_copy(rows[b], agg_sh.at[dbuf[b].at[0]], ssem[b], add=True)
            for b in range(4):
                pltpu.make_async_copy(m_hbm.at[sbuf[b].at[0]], rows[b], ssem[b]).wait()
                pltpu.async_copy(src_hbm.at[w, c + 4 + b], sbuf[b], fsem[b])
                pltpu.async_copy(dst_hbm.at[w, c + 4 + b], dbuf[b], dsem[b])
            for b in range(4):
                pltpu.make_async_copy(src_hbm.at[w, c + 4 + b], sbuf[b], fsem[b]).wait()
                pltpu.async_copy(m_hbm.at[sbuf[b].at[0]], rows[b], gsem[b])

        for b in range(4):
            pltpu.make_async_copy(m_hbm.at[sbuf[b].at[0]], rows[b], gsem[b]).wait()
            pltpu.make_async_copy(dst_hbm.at[w, 244 + b], dbuf[b], dsem[b]).wait()
            pltpu.async_copy(rows[b], agg_sh.at[dbuf[b].at[0]], ssem[b], add=True)
        for b in range(4):
            pltpu.make_async_copy(m_hbm.at[sbuf[b].at[0]], rows[b], ssem[b]).wait()

        plsc.subcore_barrier()

        pltpu.sync_copy(agg_sh.at[pl.ds(sid * _RPS, _RPS)],
                        out_hbm.at[cid, pl.ds(sid * _RPS, _RPS)])

    return k(m, src2, dst2)[:, :_N, :]


def _deg_sc(dst3):
    cp = pltpu.CompilerParams()
    if "needs_layout_passes" in pltpu.CompilerParams.__dataclass_fields__:
        cp = dataclasses.replace(cp, needs_layout_passes=False)

    @functools.partial(
        pl.kernel,
        out_type=jax.ShapeDtypeStruct((_NC, _NP), jnp.float32),
        mesh=_sc_mesh(),
        compiler_params=cp,
        scratch_types=[
            pltpu.VMEM((_EPW // _LANES, _LANES), jnp.int32),
            pltpu.VMEM((_NP,), jnp.float32),
            pltpu.VMEM((_NS, _RPS), jnp.float32),
            pltpu.VMEM_SHARED((_NS, _NP), jnp.float32),
        ],
    )
    def k(dst_hbm, out_hbm, dst_v, cnt_v, red, stage_sh):
        cid = lax.axis_index("c")
        sid = lax.axis_index("s")
        w = cid * _NS + sid

        @pl.loop(0, _NP // _LANES)
        def _(i):
            cnt_v[pl.ds(i * _LANES, _LANES)] = jnp.zeros((_LANES,), jnp.float32)

        pltpu.sync_copy(dst_hbm.at[w], dst_v)

        @pl.loop(0, _EPW // _LANES)
        def _(i):
            plsc.addupdate_scatter(cnt_v, [dst_v[i, :]],
                                   jnp.ones((_LANES,), jnp.float32))

        pltpu.sync_copy(cnt_v, stage_sh.at[sid])
        plsc.subcore_barrier()

        @pl.loop(0, _NS)
        def _(r):
            pltpu.sync_copy(stage_sh.at[r, pl.ds(sid * _RPS, _RPS)], red.at[r])

        @pl.loop(0, _RPS // _LANES)
        def _(j):
            cnt_v[pl.ds(j * _LANES, _LANES)] = jnp.zeros((_LANES,), jnp.float32)

            @pl.loop(0, _NS)
            def _(r):
                cnt_v[pl.ds(j * _LANES, _LANES)] = (
                    cnt_v[pl.ds(j * _LANES, _LANES)]
                    + red[r, pl.ds(j * _LANES, _LANES)])

        pltpu.sync_copy(cnt_v.at[pl.ds(0, _RPS)],
                        out_hbm.at[cid, pl.ds(sid * _RPS, _RPS)])

    return k(dst3)


def _tc_in(x, W_in, b_in, W0, degp):
    def body(x_ref, wi_ref, bi_ref, w0_ref, dg_ref, h_ref, m_ref, inv_ref):
        h = jax.nn.gelu(
            jnp.dot(x_ref[...], wi_ref[...], preferred_element_type=jnp.float32)
            + bi_ref[...])
        h_ref[...] = h
        m_ref[...] = jnp.dot(h, w0_ref[...], preferred_element_type=jnp.float32)
        deg = dg_ref[0] + dg_ref[1]
        inv = 1.0 / jnp.maximum(deg, 1.0)
        inv_ref[...] = jnp.broadcast_to(inv, (_N, _D))

    return pl.pallas_call(
        body,
        out_shape=(jax.ShapeDtypeStruct((_N, _D), jnp.float32),
                   jax.ShapeDtypeStruct((_N, _D), jnp.float32),
                   jax.ShapeDtypeStruct((_N, _D), jnp.float32)),
    )(x, W_in, b_in.reshape(1, _D), W0, degp)


def _tc_combine(p, inv, b, h, W):
    def body(p_ref, inv_ref, b_ref, h_ref, w_ref, hn_ref, mn_ref):
        hn = (p_ref[0] + p_ref[1]) * inv_ref[...] + b_ref[...] + h_ref[...]
        hn_ref[...] = hn
        mn_ref[...] = jnp.dot(hn, w_ref[...], preferred_element_type=jnp.float32)

    return pl.pallas_call(
        body,
        out_shape=(jax.ShapeDtypeStruct((_N, _D), jnp.float32),
                   jax.ShapeDtypeStruct((_N, _D), jnp.float32)),
    )(p, inv, b.reshape(1, _D), h, W)


def _tc_final(p, inv, b, h):
    def body(p_ref, inv_ref, b_ref, h_ref, o_ref):
        o_ref[...] = (p_ref[0] + p_ref[1]) * inv_ref[...] + b_ref[...] + h_ref[...]

    return pl.pallas_call(
        body,
        out_shape=jax.ShapeDtypeStruct((_N, _D), jnp.float32),
    )(p, inv, b.reshape(1, _D), h)


def kernel(x, edge_index, W_in, b_in, W0, b0, W1, b1, W2, b2):
    src = edge_index[0].astype(jnp.int32).reshape(_NW, _CH2, 1, _K2)
    dst = edge_index[1].astype(jnp.int32).reshape(_NW, _CH2, 1, _K2)
    dst3 = edge_index[1].astype(jnp.int32).reshape(_NW, _EPW // _LANES, _LANES)
    degp = _deg_sc(dst3)[:, :_N, None]
    h0, m0, inv = _tc_in(x, W_in, b_in, W0, degp)
    p0 = _agg_sc(m0, src, dst)
    h1, m1 = _tc_combine(p0, inv, b0, h0, W1)
    p1 = _agg_sc(m1, src, dst)
    h2, m2 = _tc_combine(p1, inv, b1, h1, W2)
    p2 = _agg_sc(m2, src, dst)
    return _tc_final(p2, inv, b2, h2)


# speedup vs baseline: 1.0752x; 1.0752x over previous
import dataclasses
import functools

import jax
import jax.numpy as jnp
from jax import lax
from jax.experimental import pallas as pl
from jax.experimental.pallas import tpu as pltpu
from jax.experimental.pallas import tpu_sc as plsc

_N = 10000
_E = 320000
_D = 128
_NC = 2
_NS = 16
_LANES = 16
_NW = _NC * _NS
_EPW = _E // _NW
_K = 80
_CHUNKS = _EPW // _K
_K2 = 40
_CH2 = _EPW // _K2
_NP = 10240
_RPS = _NP // _NS
_ZR = 128

def _sc_mesh():
    return plsc.VectorSubcoreMesh(core_axis_name="c", subcore_axis_name="s")


def _agg_sc(m, src2, dst2):
    @functools.partial(
        pl.kernel,
        out_type=jax.ShapeDtypeStruct((_NC, _NP, _D), jnp.float32),
        mesh=_sc_mesh(),
        scratch_types=[
            pltpu.VMEM((_CH2, _K2), jnp.int32),
            pltpu.VMEM((1, _K2), jnp.int32),
            pltpu.VMEM((1, _K2), jnp.int32),
            pltpu.VMEM((1, _K2), jnp.int32),
            pltpu.VMEM((_K2, _D), jnp.float32),
            pltpu.VMEM((_K2, _D), jnp.float32),
            pltpu.VMEM((_K2, _D), jnp.float32),
            pltpu.VMEM_SHARED((_NP, _D), jnp.float32),
            pltpu.SemaphoreType.DMA,
            pltpu.SemaphoreType.DMA,
            pltpu.SemaphoreType.DMA,
            pltpu.SemaphoreType.DMA,
            pltpu.SemaphoreType.DMA,
            pltpu.SemaphoreType.DMA,
            pltpu.SemaphoreType.DMA,
            pltpu.SemaphoreType.DMA,
            pltpu.SemaphoreType.DMA,
        ],
    )
    def k(m_hbm, src_hbm, dst_hbm, out_hbm,
          src_v, dA, dB, dC, r0, r1, r2, agg_sh,
          g0, g1, g2, s0, s1, s2, e0, e1, e2):
        cid = lax.axis_index("c")
        sid = lax.axis_index("s")
        w = cid * _NS + sid
        rows = (r0, r1, r2)
        dbuf = (dA, dB, dC)
        gsem = (g0, g1, g2)
        ssem = (s0, s1, s2)
        dsem = (e0, e1, e2)

        @pl.loop(0, _K2)
        def _(i):
            @pl.loop(0, _D // _LANES)
            def _(j):
                r0[i, pl.ds(j * _LANES, _LANES)] = jnp.zeros((_LANES,), jnp.float32)

        @pl.loop(0, _RPS // _K2)
        def _(t):
            pltpu.sync_copy(r0, agg_sh.at[pl.ds(sid * _RPS + t * _K2, _K2)])

        pltpu.sync_copy(src_hbm.at[w], src_v)

        plsc.subcore_barrier()

        pltpu.async_copy(m_hbm.at[src_v.at[_CH2 - 1]], r0, g0).wait()
        pltpu.sync_copy(dst_hbm.at[w, _CH2 - 1], dA)
        pltpu.sync_copy(r0, agg_sh.at[dA.at[0]], add=True)

        for b in range(3):
            pltpu.async_copy(m_hbm.at[src_v.at[b]], rows[b], gsem[b])
            pltpu.async_copy(dst_hbm.at[w, b], dbuf[b], dsem[b])

        @pl.loop(0, 82)
        def _(i):
            c = 3 * i
            for b in range(3):
                pltpu.make_async_copy(m_hbm.at[src_v.at[c + b]], rows[b], gsem[b]).wait()
                pltpu.make_async_copy(dst_hbm.at[w, c + b], dbuf[b], dsem[b]).wait()
                pltpu.async_copy(rows[b], agg_sh.at[dbuf[b].at[0]], ssem[b], add=True)
            for b in range(3):
                pltpu.make_async_copy(m_hbm.at[src_v.at[c + b]], rows[b], ssem[b]).wait()
                pltpu.async_copy(m_hbm.at[src_v.at[c + 3 + b]], rows[b], gsem[b])
                pltpu.async_copy(dst_hbm.at[w, c + 3 + b], dbuf[b], dsem[b])

        for b in range(3):
            pltpu.make_async_copy(m_hbm.at[src_v.at[246 + b]], rows[b], gsem[b]).wait()
            pltpu.make_async_copy(dst_hbm.at[w, 246 + b], dbuf[b], dsem[b]).wait()
            pltpu.async_copy(rows[b], agg_sh.at[dbuf[b].at[0]], ssem[b], add=True)
        for b in range(3):
            pltpu.make_async_copy(m_hbm.at[src_v.at[246 + b]], rows[b], ssem[b]).wait()

        plsc.subcore_barrier()

        pltpu.sync_copy(agg_sh.at[pl.ds(sid * _RPS, _RPS)],
                        out_hbm.at[cid, pl.ds(sid * _RPS, _RPS)])

    return k(m, src2, dst2)[:, :_N, :]


def _deg_sc(dst3):
    cp = pltpu.CompilerParams()
    if "needs_layout_passes" in pltpu.CompilerParams.__dataclass_fields__:
        cp = dataclasses.replace(cp, needs_layout_passes=False)

    @functools.partial(
        pl.kernel,
        out_type=jax.ShapeDtypeStruct((_NC, _NP), jnp.float32),
        mesh=_sc_mesh(),
        compiler_params=cp,
        scratch_types=[
            pltpu.VMEM((_EPW // _LANES, _LANES), jnp.int32),
            pltpu.VMEM((_NP,), jnp.float32),
            pltpu.VMEM((_NS, _RPS), jnp.float32),
            pltpu.VMEM_SHARED((_NS, _NP), jnp.float32),
        ],
    )
    def k(dst_hbm, out_hbm, dst_v, cnt_v, red, stage_sh):
        cid = lax.axis_index("c")
        sid = lax.axis_index("s")
        w = cid * _NS + sid

        @pl.loop(0, _NP // _LANES)
        def _(i):
            cnt_v[pl.ds(i * _LANES, _LANES)] = jnp.zeros((_LANES,), jnp.float32)

        pltpu.sync_copy(dst_hbm.at[w], dst_v)

        @pl.loop(0, _EPW // _LANES)
        def _(i):
            plsc.addupdate_scatter(cnt_v, [dst_v[i, :]],
                                   jnp.ones((_LANES,), jnp.float32))

        pltpu.sync_copy(cnt_v, stage_sh.at[sid])
        plsc.subcore_barrier()

        @pl.loop(0, _NS)
        def _(r):
            pltpu.sync_copy(stage_sh.at[r, pl.ds(sid * _RPS, _RPS)], red.at[r])

        @pl.loop(0, _RPS // _LANES)
        def _(j):
            cnt_v[pl.ds(j * _LANES, _LANES)] = jnp.zeros((_LANES,), jnp.float32)

            @pl.loop(0, _NS)
            def _(r):
                cnt_v[pl.ds(j * _LANES, _LANES)] = (
                    cnt_v[pl.ds(j * _LANES, _LANES)]
                    + red[r, pl.ds(j * _LANES, _LANES)])

        pltpu.sync_copy(cnt_v.at[pl.ds(0, _RPS)],
                        out_hbm.at[cid, pl.ds(sid * _RPS, _RPS)])

    return k(dst3)


def _tc_in(x, W_in, b_in, W0, degp):
    def body(x_ref, wi_ref, bi_ref, w0_ref, dg_ref, h_ref, m_ref, inv_ref):
        h = jax.nn.gelu(
            jnp.dot(x_ref[...], wi_ref[...], preferred_element_type=jnp.float32)
            + bi_ref[...])
        h_ref[...] = h
        m_ref[...] = jnp.dot(h, w0_ref[...], preferred_element_type=jnp.float32)
        deg = dg_ref[0] + dg_ref[1]
        inv = 1.0 / jnp.maximum(deg, 1.0)
        inv_ref[...] = jnp.broadcast_to(inv, (_N, _D))

    return pl.pallas_call(
        body,
        out_shape=(jax.ShapeDtypeStruct((_N, _D), jnp.float32),
                   jax.ShapeDtypeStruct((_N, _D), jnp.float32),
                   jax.ShapeDtypeStruct((_N, _D), jnp.float32)),
    )(x, W_in, b_in.reshape(1, _D), W0, degp)


def _tc_combine(p, inv, b, h, W):
    def body(p_ref, inv_ref, b_ref, h_ref, w_ref, hn_ref, mn_ref):
        hn = (p_ref[0] + p_ref[1]) * inv_ref[...] + b_ref[...] + h_ref[...]
        hn_ref[...] = hn
        mn_ref[...] = jnp.dot(hn, w_ref[...], preferred_element_type=jnp.float32)

    return pl.pallas_call(
        body,
        out_shape=(jax.ShapeDtypeStruct((_N, _D), jnp.float32),
                   jax.ShapeDtypeStruct((_N, _D), jnp.float32)),
    )(p, inv, b.reshape(1, _D), h, W)


def _tc_final(p, inv, b, h):
    def body(p_ref, inv_ref, b_ref, h_ref, o_ref):
        o_ref[...] = (p_ref[0] + p_ref[1]) * inv_ref[...] + b_ref[...] + h_ref[...]

    return pl.pallas_call(
        body,
        out_shape=jax.ShapeDtypeStruct((_N, _D), jnp.float32),
    )(p, inv, b.reshape(1, _D), h)


def kernel(x, edge_index, W_in, b_in, W0, b0, W1, b1, W2, b2):
    src = edge_index[0].astype(jnp.int32).reshape(_NW, _CH2, _K2)
    dst = edge_index[1].astype(jnp.int32).reshape(_NW, _CH2, 1, _K2)
    dst3 = edge_index[1].astype(jnp.int32).reshape(_NW, _EPW // _LANES, _LANES)
    degp = _deg_sc(dst3)[:, :_N, None]
    h0, m0, inv = _tc_in(x, W_in, b_in, W0, degp)
    p0 = _agg_sc(m0, src, dst)
    h1, m1 = _tc_combine(p0, inv, b0, h0, W1)
    p1 = _agg_sc(m1, src, dst)
    h2, m2 = _tc_combine(p1, inv, b1, h1, W2)
    p2 = _agg_sc(m2, src, dst)
    return _tc_final(p2, inv, b2, h2)
